# baseline (speedup 1.0000x reference)
import jax
import jax.numpy as jnp
from jax import lax
from jax.experimental import pallas as pl
from jax.experimental.pallas import tpu as pltpu
from jax.experimental.pallas import tpu_sc as plsc

N = 10000
D = 128
E = 320000
K_APPNP = 5
ALPHA = 0.2

NC = 2
NS = 16
L = 16

NP = 10112
RPT = NP // NS

EPW = E // (NC * NS)
CHUNK = 112
NBUF = 2
NCHUNK = 90
DEAD = NCHUNK
EPWP = (NCHUNK + 1) * CHUNK

_MESH = plsc.VectorSubcoreMesh(
    core_axis_name="c", subcore_axis_name="s", num_cores=NC, num_subcores=NS
)
_SC_PARAMS = pltpu.CompilerParams(
    use_tc_tiling_on_sc=False, needs_layout_passes=False
)


def _sc_deg_body(row_hbm, col_hbm, out_hbm, row_v, col_v, hist):
    c = lax.axis_index("c")
    s = lax.axis_index("s")
    w = s * NC + c
    base = w * EPW
    pltpu.sync_copy(row_hbm.at[pl.ds(base, EPW)], row_v)
    pltpu.sync_copy(col_hbm.at[pl.ds(base, EPW)], col_v)

    zeros = jnp.zeros((L,), jnp.float32)

    def zb(i, carry):
        for j in range(3):
            hist[j, pl.ds(i * L, L)] = zeros
        return carry

    lax.fori_loop(0, NP // L, zb, 0)

    ones = jnp.ones((L,), jnp.float32)
    r0 = jnp.zeros((L,), jnp.int32)
    r1 = jnp.full((L,), 1, jnp.int32)
    r2 = jnp.full((L,), 2, jnp.int32)

    def eb(i, carry):
        r = row_v[pl.ds(i * L, L)]
        cl = col_v[pl.ds(i * L, L)]
        plsc.addupdate_scatter(hist, [r0, r], ones)
        plsc.addupdate_scatter(hist, [r1, cl], ones)
        plsc.addupdate_scatter(hist, [r2, cl], ones, mask=r == cl)
        return carry

    lax.fori_loop(0, EPW // L, eb, 0)

    pltpu.sync_copy(hist, out_hbm.at[pl.ds(w * 8, 8)])


_sc_deg = pl.kernel(
    _sc_deg_body,
    out_type=jax.ShapeDtypeStruct((NC * NS * 8, NP), jnp.float32),
    mesh=_MESH,
    scratch_types=[
        pltpu.VMEM((EPW,), jnp.int32),
        pltpu.VMEM((EPW,), jnp.int32),
        pltpu.VMEM((8, NP), jnp.float32),
    ],
    compiler_params=_SC_PARAMS,
)


def _sc_prop_body(h_hbm, rowi_hbm, coli_hbm, zslab_hbm, out_hbm,
                  acc_sh, row_v, col_v, gb0, gb1, sg0, sg1, ss0, ss1):
    c = lax.axis_index("c")
    s = lax.axis_index("s")
    gbufs = (gb0, gb1)
    semg = (sg0, sg1)
    sems = (ss0, ss1)

    pltpu.sync_copy(rowi_hbm.at[c, s], row_v)
    pltpu.sync_copy(coli_hbm.at[c, s], col_v)
    pltpu.sync_copy(zslab_hbm, acc_sh.at[pl.ds(s * RPT, RPT)])
    plsc.subcore_barrier()

    pltpu.async_copy(gb0, acc_sh.at[col_v.at[DEAD]], ss0, add=True)
    pltpu.async_copy(h_hbm.at[row_v.at[DEAD]], gb1, sg1)

    def body(q, carry):
        for b in range(NBUF):
            g = NBUF * q + b
            b2 = 1 - b
            pltpu.make_async_copy(gbufs[b], acc_sh.at[col_v.at[DEAD]],
                                  sems[b]).wait()
            pltpu.async_copy(h_hbm.at[row_v.at[g]], gbufs[b], semg[b])
            pltpu.make_async_copy(h_hbm.at[row_v.at[DEAD]], gbufs[b2],
                                  semg[b2]).wait()
            sci = jnp.where(g == 0, DEAD, g - 1)
            pltpu.async_copy(gbufs[b2], acc_sh.at[col_v.at[sci]],
                             sems[b2], add=True)
        return carry

    lax.fori_loop(0, NCHUNK // NBUF, body, 0)

    pltpu.make_async_copy(h_hbm.at[row_v.at[DEAD]], gb1, sg1).wait()
    pltpu.async_copy(gb1, acc_sh.at[col_v.at[NCHUNK - 1]], ss1, add=True)
    pltpu.make_async_copy(gb0, acc_sh.at[col_v.at[DEAD]], ss0).wait()
    pltpu.make_async_copy(gb1, acc_sh.at[col_v.at[DEAD]], ss1).wait()
    plsc.subcore_barrier()

    pltpu.sync_copy(
        acc_sh.at[pl.ds(s * RPT, RPT)],
        out_hbm.at[pl.ds(c * NP + s * RPT, RPT)],
    )


_sc_prop = pl.kernel(
    _sc_prop_body,
    out_type=jax.ShapeDtypeStruct((2 * NP, D), jnp.float32),
    mesh=_MESH,
    scratch_types=[
        pltpu.VMEM_SHARED((NP, D), jnp.float32),
        pltpu.VMEM((NCHUNK + 1, CHUNK), jnp.int32),
        pltpu.VMEM((NCHUNK + 1, CHUNK), jnp.int32),
        pltpu.VMEM((CHUNK, D), jnp.float32),
        pltpu.VMEM((CHUNK, D), jnp.float32),
        pltpu.SemaphoreType.DMA,
        pltpu.SemaphoreType.DMA,
        pltpu.SemaphoreType.DMA,
        pltpu.SemaphoreType.DMA,
    ],
    compiler_params=_SC_PARAMS,
)


def _tc_prep_body(part_ref, norms_ref):
    ps = jnp.sum(part_ref[...].reshape(NC * NS, 8, NP)[:, :3], axis=0)
    deg_out = ps[0]
    deg_in = ps[1]
    selfc = ps[2]
    src = lax.rsqrt(jnp.maximum(deg_out, 1.0))
    dst_eff = (1.0 - ALPHA) * lax.rsqrt(jnp.maximum(deg_in, 1.0))
    cgc_inv = 1.0 / jnp.maximum(deg_in - selfc + 1.0, 1.0)
    norms_ref[...] = jnp.stack([src, dst_eff, cgc_inv, selfc])


_tc_prep = pl.pallas_call(
    _tc_prep_body,
    out_shape=jax.ShapeDtypeStruct((4, NP), jnp.float32),
)


def _unstack(acc_ref):
    return acc_ref[0:N, :] + acc_ref[NP:NP + N, :]


def _tc_pre_body(x_ref, nt_ref, hs_ref):
    hs_ref[...] = x_ref[...] * nt_ref[:, 0:1]


_tc_pre = pl.pallas_call(
    _tc_pre_body,
    out_shape=jax.ShapeDtypeStruct((N, D), jnp.float32),
)


def _tc_mid_body(acc_ref, x0_ref, nt_ref, hs_ref):
    h = _unstack(acc_ref) * nt_ref[:, 1:2] + ALPHA * x0_ref[...]
    hs_ref[...] = h * nt_ref[:, 0:1]


_tc_mid = pl.pallas_call(
    _tc_mid_body,
    out_shape=jax.ShapeDtypeStruct((N, D), jnp.float32),
)


def _tc_last_body(acc_ref, x0_ref, nt_ref, h_ref):
    h_ref[...] = _unstack(acc_ref) * nt_ref[:, 1:2] + ALPHA * x0_ref[...]


_tc_last = pl.pallas_call(
    _tc_last_body,
    out_shape=jax.ShapeDtypeStruct((N, D), jnp.float32),
)


def _mish(z):
    return z * jnp.tanh(jax.nn.softplus(z))


def _cgc(acc_ref, h_ref, nt_ref, wo_ref, bo_ref, wr_ref):
    h = h_ref[...]
    agg = nt_ref[:, 2:3] * (_unstack(acc_ref) + (1.0 - nt_ref[:, 3:4]) * h)
    return (jnp.dot(agg, wo_ref[...], preferred_element_type=jnp.float32)
            + bo_ref[...][None, :]
            + jnp.dot(h, wr_ref[...], preferred_element_type=jnp.float32))


def _tc_cgc0_body(acc_ref, h_ref, nt_ref, wo_ref, bo_ref, wr_ref, out_ref):
    out_ref[...] = _cgc(acc_ref, h_ref, nt_ref, wo_ref, bo_ref, wr_ref)


_tc_cgc0 = pl.pallas_call(
    _tc_cgc0_body,
    out_shape=jax.ShapeDtypeStruct((N, D), jnp.float32),
)


def _tc_cgc1_body(acc_ref, h_ref, nt_ref, wo_ref, bo_ref, wr_ref, out_ref):
    out_ref[...] = _mish(_cgc(acc_ref, h_ref, nt_ref, wo_ref, bo_ref, wr_ref))


_tc_cgc1 = pl.pallas_call(
    _tc_cgc1_body,
    out_shape=jax.ShapeDtypeStruct((N, D), jnp.float32),
)


def kernel(x, edge_index, W_out0, b_out0, W_root0, W_out1, b_out1, W_root1):
    row = edge_index[0].astype(jnp.int32)
    col = edge_index[1].astype(jnp.int32)

    row2 = jnp.pad(row.reshape(NC * NS, EPW), ((0, 0), (0, EPWP - EPW)))
    col2 = jnp.pad(col.reshape(NC * NS, EPW), ((0, 0), (0, EPWP - EPW)),
                   constant_values=N)
    rowi = row2.reshape(NS, NC, NCHUNK + 1, CHUNK).transpose(1, 0, 2, 3)
    coli = col2.reshape(NS, NC, NCHUNK + 1, CHUNK).transpose(1, 0, 2, 3)
    zslab = jnp.zeros((RPT, D), jnp.float32)

    partials = _sc_deg(row, col)
    norms = _tc_prep(partials)
    nt = jnp.transpose(norms[:, :N])

    hs = _tc_pre(x, nt)
    for _ in range(K_APPNP - 1):
        acc = _sc_prop(hs, rowi, coli, zslab)
        hs = _tc_mid(acc, x, nt)
    acc = _sc_prop(hs, rowi, coli, zslab)
    h = _tc_last(acc, x, nt)

    acc = _sc_prop(h, rowi, coli, zslab)
    h = _tc_cgc0(acc, h, nt, W_out0, b_out0, W_root0)
    acc = _sc_prop(h, rowi, coli, zslab)
    return _tc_cgc1(acc, h, nt, W_out1, b_out1, W_root1)


# speedup vs baseline: 1.6265x; 1.6265x over previous
import jax
import jax.numpy as jnp
from jax import lax
from jax.experimental import pallas as pl
from jax.experimental.pallas import tpu as pltpu
from jax.experimental.pallas import tpu_sc as plsc

N = 10000
D = 128
E = 320000
K_APPNP = 5
ALPHA = 0.2

NC = 2
NS = 16
L = 16
HALF = D // NC

NP = 10112
RPT = NP // NS

EPT = E // NS
CHUNK = 112
NBUF = 2
NCHUNK = 180
EPTP = NCHUNK * CHUNK

EPW = E // (NC * NS)

_MESH = plsc.VectorSubcoreMesh(
    core_axis_name="c", subcore_axis_name="s", num_cores=NC, num_subcores=NS
)
_SC_PARAMS = pltpu.CompilerParams(
    use_tc_tiling_on_sc=False, needs_layout_passes=False
)


def _sc_deg_body(row_hbm, col_hbm, out_hbm, row_v, col_v, hist):
    c = lax.axis_index("c")
    s = lax.axis_index("s")
    w = s * NC + c
    base = w * EPW
    pltpu.sync_copy(row_hbm.at[pl.ds(base, EPW)], row_v)
    pltpu.sync_copy(col_hbm.at[pl.ds(base, EPW)], col_v)

    zeros = jnp.zeros((L,), jnp.float32)

    def zb(i, carry):
        for j in range(3):
            hist[j, pl.ds(i * L, L)] = zeros
        return carry

    lax.fori_loop(0, NP // L, zb, 0)

    ones = jnp.ones((L,), jnp.float32)
    r0 = jnp.zeros((L,), jnp.int32)
    r1 = jnp.full((L,), 1, jnp.int32)
    r2 = jnp.full((L,), 2, jnp.int32)

    def eb(i, carry):
        r = row_v[pl.ds(i * L, L)]
        cl = col_v[pl.ds(i * L, L)]
        plsc.addupdate_scatter(hist, [r0, r], ones)
        plsc.addupdate_scatter(hist, [r1, cl], ones)
        plsc.addupdate_scatter(hist, [r2, cl], ones, mask=r == cl)
        return carry

    lax.fori_loop(0, EPW // L, eb, 0)

    pltpu.sync_copy(hist, out_hbm.at[pl.ds(w * 8, 8)])


_sc_deg = pl.kernel(
    _sc_deg_body,
    out_type=jax.ShapeDtypeStruct((NC * NS * 8, NP), jnp.float32),
    mesh=_MESH,
    scratch_types=[
        pltpu.VMEM((EPW,), jnp.int32),
        pltpu.VMEM((EPW,), jnp.int32),
        pltpu.VMEM((8, NP), jnp.float32),
    ],
    compiler_params=_SC_PARAMS,
)


def _sc_prop_body(h_hbm, pki_hbm, zslab_hbm, out_hbm,
                  h_sp, acc_sh, pk_v, rb0, cb0, rb1, cb1,
                  gb0, gb1, sg0, sg1, ss0, ss1):
    c = lax.axis_index("c")
    s = lax.axis_index("s")
    rbufs = (rb0, rb1)
    cbufs = (cb0, cb1)
    gbufs = (gb0, gb1)
    semg = (sg0, sg1)
    sems = (ss0, ss1)

    pltpu.sync_copy(pki_hbm.at[s], pk_v)
    pltpu.sync_copy(h_hbm.at[pl.ds(s * RPT, RPT), pl.ds(c * HALF, HALF)],
                    h_sp.at[pl.ds(s * RPT, RPT)])
    pltpu.sync_copy(zslab_hbm, acc_sh.at[pl.ds(s * RPT, RPT)])
    plsc.subcore_barrier()

    mask14 = jnp.full((L,), 0x3FFF, jnp.int32)
    sh14 = jnp.full((L,), 14, jnp.int32)

    def unpack(g, b):
        for i in range(CHUNK // L):
            v = pk_v[g, pl.ds(i * L, L)]
            rbufs[b][pl.ds(i * L, L)] = v & mask14
            cbufs[b][pl.ds(i * L, L)] = lax.shift_right_logical(v, sh14)

    def body(q, carry):
        g = NBUF * q
        cps = []
        for b in range(NBUF):
            unpack(g + b, b)
            cps.append(pltpu.async_copy(h_sp.at[rbufs[b]], gbufs[b], semg[b]))
        scs = []
        for b in range(NBUF):
            cps[b].wait()
            scs.append(
                pltpu.async_copy(gbufs[b], acc_sh.at[cbufs[b]],
                                 sems[b], add=True)
            )
        for b in range(NBUF):
            scs[b].wait()
        return carry

    lax.fori_loop(0, NCHUNK // NBUF, body, 0)
    plsc.subcore_barrier()

    pltpu.sync_copy(
        acc_sh.at[pl.ds(s * RPT, RPT)],
        out_hbm.at[pl.ds(s * RPT, RPT), pl.ds(c * HALF, HALF)],
    )


_sc_prop = pl.kernel(
    _sc_prop_body,
    out_type=jax.ShapeDtypeStruct((NP, D), jnp.float32),
    mesh=_MESH,
    scratch_types=[
        pltpu.VMEM_SHARED((NP, HALF), jnp.float32),
        pltpu.VMEM_SHARED((NP, HALF), jnp.float32),
        pltpu.VMEM((NCHUNK, CHUNK), jnp.int32),
        pltpu.VMEM((CHUNK,), jnp.int32),
        pltpu.VMEM((CHUNK,), jnp.int32),
        pltpu.VMEM((CHUNK,), jnp.int32),
        pltpu.VMEM((CHUNK,), jnp.int32),
        pltpu.VMEM((CHUNK, HALF), jnp.float32),
        pltpu.VMEM((CHUNK, HALF), jnp.float32),
        pltpu.SemaphoreType.DMA,
        pltpu.SemaphoreType.DMA,
        pltpu.SemaphoreType.DMA,
        pltpu.SemaphoreType.DMA,
    ],
    compiler_params=_SC_PARAMS,
)


def _tc_prep_body(part_ref, norms_ref):
    ps = jnp.sum(part_ref[...].reshape(NC * NS, 8, NP)[:, :3], axis=0)
    deg_out = ps[0]
    deg_in = ps[1]
    selfc = ps[2]
    src = lax.rsqrt(jnp.maximum(deg_out, 1.0))
    dst_eff = (1.0 - ALPHA) * lax.rsqrt(jnp.maximum(deg_in, 1.0))
    cgc_inv = 1.0 / jnp.maximum(deg_in - selfc + 1.0, 1.0)
    norms_ref[...] = jnp.stack([src, dst_eff, cgc_inv, selfc])


_tc_prep = pl.pallas_call(
    _tc_prep_body,
    out_shape=jax.ShapeDtypeStruct((4, NP), jnp.float32),
)


def _tc_pre_body(x_ref, nt_ref, hs_ref):
    hs_ref[0:N, :] = x_ref[...] * nt_ref[:, 0:1]


_tc_pre = pl.pallas_call(
    _tc_pre_body,
    out_shape=jax.ShapeDtypeStruct((NP, D), jnp.float32),
)


def _tc_mid_body(acc_ref, x0_ref, nt_ref, hs_ref):
    h = acc_ref[0:N, :] * nt_ref[:, 1:2] + ALPHA * x0_ref[...]
    hs_ref[0:N, :] = h * nt_ref[:, 0:1]


_tc_mid = pl.pallas_call(
    _tc_mid_body,
    out_shape=jax.ShapeDtypeStruct((NP, D), jnp.float32),
)


def _tc_last_body(acc_ref, x0_ref, nt_ref, h_ref):
    h_ref[0:N, :] = acc_ref[0:N, :] * nt_ref[:, 1:2] + ALPHA * x0_ref[...]


_tc_last = pl.pallas_call(
    _tc_last_body,
    out_shape=jax.ShapeDtypeStruct((NP, D), jnp.float32),
)


def _mish(z):
    return z * jnp.tanh(jax.nn.softplus(z))


def _cgc(acc_ref, h_ref, nt_ref, wo_ref, bo_ref, wr_ref):
    h = h_ref[0:N, :]
    agg = nt_ref[:, 2:3] * (acc_ref[0:N, :] + (1.0 - nt_ref[:, 3:4]) * h)
    return (jnp.dot(agg, wo_ref[...], preferred_element_type=jnp.float32)
            + bo_ref[...][None, :]
            + jnp.dot(h, wr_ref[...], preferred_element_type=jnp.float32))


def _tc_cgc0_body(acc_ref, h_ref, nt_ref, wo_ref, bo_ref, wr_ref, out_ref):
    out_ref[0:N, :] = _cgc(acc_ref, h_ref, nt_ref, wo_ref, bo_ref, wr_ref)


_tc_cgc0 = pl.pallas_call(
    _tc_cgc0_body,
    out_shape=jax.ShapeDtypeStruct((NP, D), jnp.float32),
)


def _tc_cgc1_body(acc_ref, h_ref, nt_ref, wo_ref, bo_ref, wr_ref, out_ref):
    out_ref[...] = _mish(_cgc(acc_ref, h_ref, nt_ref, wo_ref, bo_ref, wr_ref))


_tc_cgc1 = pl.pallas_call(
    _tc_cgc1_body,
    out_shape=jax.ShapeDtypeStruct((N, D), jnp.float32),
)


def kernel(x, edge_index, W_out0, b_out0, W_root0, W_out1, b_out1, W_root1):
    row = edge_index[0].astype(jnp.int32)
    col = edge_index[1].astype(jnp.int32)

    packed = row | (col << 14)
    pki = jnp.pad(packed.reshape(NS, EPT), ((0, 0), (0, EPTP - EPT)),
                  constant_values=N << 14).reshape(NS, NCHUNK, CHUNK)
    zslab = jnp.zeros((RPT, HALF), jnp.float32)

    partials = _sc_deg(row, col)
    norms = _tc_prep(partials)
    nt = jnp.transpose(norms[:, :N])

    hs = _tc_pre(x, nt)
    for _ in range(K_APPNP - 1):
        acc = _sc_prop(hs, pki, zslab)
        hs = _tc_mid(acc, x, nt)
    acc = _sc_prop(hs, pki, zslab)
    h = _tc_last(acc, x, nt)

    acc = _sc_prop(h, pki, zslab)
    h = _tc_cgc0(acc, h, nt, W_out0, b_out0, W_root0)
    acc = _sc_prop(h, pki, zslab)
    return _tc_cgc1(acc, h, nt, W_out1, b_out1, W_root1)


# speedup vs baseline: 2.1877x; 1.3450x over previous
import jax
import jax.numpy as jnp
from jax import lax
from jax.experimental import pallas as pl
from jax.experimental.pallas import tpu as pltpu
from jax.experimental.pallas import tpu_sc as plsc

N = 10000
D = 128
E = 320000
K_APPNP = 5
ALPHA = 0.2

NC = 2
NS = 16
L = 16
HALF = D // NC

NP = 10112
RPT = NP // NS

EPT = E // NS
CHUNK = 128
NBUF = 3
NCHUNK = 159
EPTP = NCHUNK * CHUNK

EPW = E // (NC * NS)

_MESH = plsc.VectorSubcoreMesh(
    core_axis_name="c", subcore_axis_name="s", num_cores=NC, num_subcores=NS
)
_SC_PARAMS = pltpu.CompilerParams(
    use_tc_tiling_on_sc=False, needs_layout_passes=False
)


def _sc_deg_body(row_hbm, col_hbm, out_hbm, row_v, col_v, hist):
    c = lax.axis_index("c")
    s = lax.axis_index("s")
    w = s * NC + c
    base = w * EPW
    pltpu.sync_copy(row_hbm.at[pl.ds(base, EPW)], row_v)
    pltpu.sync_copy(col_hbm.at[pl.ds(base, EPW)], col_v)

    zeros = jnp.zeros((L,), jnp.float32)

    def zb(i, carry):
        for j in range(3):
            hist[j, pl.ds(i * L, L)] = zeros
        return carry

    lax.fori_loop(0, NP // L, zb, 0)

    ones = jnp.ones((L,), jnp.float32)
    r0 = jnp.zeros((L,), jnp.int32)
    r1 = jnp.full((L,), 1, jnp.int32)
    r2 = jnp.full((L,), 2, jnp.int32)

    def eb(i, carry):
        r = row_v[pl.ds(i * L, L)]
        cl = col_v[pl.ds(i * L, L)]
        plsc.addupdate_scatter(hist, [r0, r], ones)
        plsc.addupdate_scatter(hist, [r1, cl], ones)
        plsc.addupdate_scatter(hist, [r2, cl], ones, mask=r == cl)
        return carry

    lax.fori_loop(0, EPW // L, eb, 0)

    pltpu.sync_copy(hist, out_hbm.at[pl.ds(w * 8, 8)])


_sc_deg = pl.kernel(
    _sc_deg_body,
    out_type=jax.ShapeDtypeStruct((NC * NS * 8, NP), jnp.float32),
    mesh=_MESH,
    scratch_types=[
        pltpu.VMEM((EPW,), jnp.int32),
        pltpu.VMEM((EPW,), jnp.int32),
        pltpu.VMEM((8, NP), jnp.float32),
    ],
    compiler_params=_SC_PARAMS,
)


def _sc_prop_body(h_hbm, pki_hbm, zslab_hbm, out_hbm,
                  h_sp, acc_sh, pk_v, rb0, cb0, rb1, cb1, rb2, cb2,
                  gb0, gb1, gb2, sg0, sg1, sg2, ss0, ss1, ss2):
    c = lax.axis_index("c")
    s = lax.axis_index("s")
    rbufs = (rb0, rb1, rb2)
    cbufs = (cb0, cb1, cb2)
    gbufs = (gb0, gb1, gb2)
    semg = (sg0, sg1, sg2)
    sems = (ss0, ss1, ss2)

    pltpu.sync_copy(pki_hbm.at[s], pk_v)
    pltpu.sync_copy(h_hbm.at[pl.ds(s * RPT, RPT), pl.ds(c * HALF, HALF)],
                    h_sp.at[pl.ds(s * RPT, RPT)])
    pltpu.sync_copy(zslab_hbm, acc_sh.at[pl.ds(s * RPT, RPT)])
    plsc.subcore_barrier()

    mask14 = jnp.full((L,), 0x3FFF, jnp.int32)
    sh14 = jnp.full((L,), 14, jnp.int32)

    def unpack(g, b):
        for i in range(CHUNK // L):
            v = pk_v[g, pl.ds(i * L, L)]
            rbufs[b][pl.ds(i * L, L)] = v & mask14
            cbufs[b][pl.ds(i * L, L)] = lax.shift_right_logical(v, sh14)

    def body(q, carry):
        g = NBUF * q
        cps = []
        for b in range(NBUF):
            unpack(g + b, b)
            cps.append(pltpu.async_copy(h_sp.at[rbufs[b]], gbufs[b], semg[b]))
        scs = []
        for b in range(NBUF):
            cps[b].wait()
            scs.append(
                pltpu.async_copy(gbufs[b], acc_sh.at[cbufs[b]],
                                 sems[b], add=True)
            )
        for b in range(NBUF):
            scs[b].wait()
        return carry

    lax.fori_loop(0, NCHUNK // NBUF, body, 0)
    plsc.subcore_barrier()

    pltpu.sync_copy(
        acc_sh.at[pl.ds(s * RPT, RPT)],
        out_hbm.at[pl.ds(s * RPT, RPT), pl.ds(c * HALF, HALF)],
    )


_sc_prop = pl.kernel(
    _sc_prop_body,
    out_type=jax.ShapeDtypeStruct((NP, D), jnp.float32),
    mesh=_MESH,
    scratch_types=[
        pltpu.VMEM_SHARED((NP, HALF), jnp.float32),
        pltpu.VMEM_SHARED((NP, HALF), jnp.float32),
        pltpu.VMEM((NCHUNK, CHUNK), jnp.int32),
        pltpu.VMEM((CHUNK,), jnp.int32),
        pltpu.VMEM((CHUNK,), jnp.int32),
        pltpu.VMEM((CHUNK,), jnp.int32),
        pltpu.VMEM((CHUNK,), jnp.int32),
        pltpu.VMEM((CHUNK,), jnp.int32),
        pltpu.VMEM((CHUNK,), jnp.int32),
        pltpu.VMEM((CHUNK, HALF), jnp.float32),
        pltpu.VMEM((CHUNK, HALF), jnp.float32),
        pltpu.VMEM((CHUNK, HALF), jnp.float32),
        pltpu.SemaphoreType.DMA,
        pltpu.SemaphoreType.DMA,
        pltpu.SemaphoreType.DMA,
        pltpu.SemaphoreType.DMA,
        pltpu.SemaphoreType.DMA,
        pltpu.SemaphoreType.DMA,
    ],
    compiler_params=_SC_PARAMS,
)


def _tc_prep_body(part_ref, norms_ref):
    ps = jnp.sum(part_ref[...].reshape(NC * NS, 8, NP)[:, :3], axis=0)
    deg_out = ps[0]
    deg_in = ps[1]
    selfc = ps[2]
    src = lax.rsqrt(jnp.maximum(deg_out, 1.0))
    dst_eff = (1.0 - ALPHA) * lax.rsqrt(jnp.maximum(deg_in, 1.0))
    cgc_inv = 1.0 / jnp.maximum(deg_in - selfc + 1.0, 1.0)
    norms_ref[...] = jnp.stack([src, dst_eff, cgc_inv, selfc])


_tc_prep = pl.pallas_call(
    _tc_prep_body,
    out_shape=jax.ShapeDtypeStruct((4, NP), jnp.float32),
)


def _tc_pre_body(x_ref, nt_ref, hs_ref):
    hs_ref[0:N, :] = x_ref[...] * nt_ref[:, 0:1]


_tc_pre = pl.pallas_call(
    _tc_pre_body,
    out_shape=jax.ShapeDtypeStruct((NP, D), jnp.float32),
)


def _tc_mid_body(acc_ref, x0_ref, nt_ref, hs_ref):
    h = acc_ref[0:N, :] * nt_ref[:, 1:2] + ALPHA * x0_ref[...]
    hs_ref[0:N, :] = h * nt_ref[:, 0:1]


_tc_mid = pl.pallas_call(
    _tc_mid_body,
    out_shape=jax.ShapeDtypeStruct((NP, D), jnp.float32),
)


def _tc_last_body(acc_ref, x0_ref, nt_ref, h_ref):
    h_ref[0:N, :] = acc_ref[0:N, :] * nt_ref[:, 1:2] + ALPHA * x0_ref[...]


_tc_last = pl.pallas_call(
    _tc_last_body,
    out_shape=jax.ShapeDtypeStruct((NP, D), jnp.float32),
)


def _mish(z):
    return z * jnp.tanh(jax.nn.softplus(z))


def _cgc(acc_ref, h_ref, nt_ref, wo_ref, bo_ref, wr_ref):
    h = h_ref[0:N, :]
    agg = nt_ref[:, 2:3] * (acc_ref[0:N, :] + (1.0 - nt_ref[:, 3:4]) * h)
    return (jnp.dot(agg, wo_ref[...], preferred_element_type=jnp.float32)
            + bo_ref[...][None, :]
            + jnp.dot(h, wr_ref[...], preferred_element_type=jnp.float32))


def _tc_cgc0_body(acc_ref, h_ref, nt_ref, wo_ref, bo_ref, wr_ref, out_ref):
    out_ref[0:N, :] = _cgc(acc_ref, h_ref, nt_ref, wo_ref, bo_ref, wr_ref)


_tc_cgc0 = pl.pallas_call(
    _tc_cgc0_body,
    out_shape=jax.ShapeDtypeStruct((NP, D), jnp.float32),
)


def _tc_cgc1_body(acc_ref, h_ref, nt_ref, wo_ref, bo_ref, wr_ref, out_ref):
    out_ref[...] = _mish(_cgc(acc_ref, h_ref, nt_ref, wo_ref, bo_ref, wr_ref))


_tc_cgc1 = pl.pallas_call(
    _tc_cgc1_body,
    out_shape=jax.ShapeDtypeStruct((N, D), jnp.float32),
)


def kernel(x, edge_index, W_out0, b_out0, W_root0, W_out1, b_out1, W_root1):
    row = edge_index[0].astype(jnp.int32)
    col = edge_index[1].astype(jnp.int32)

    packed = row | (col << 14)
    pki = jnp.pad(packed.reshape(NS, EPT), ((0, 0), (0, EPTP - EPT)),
                  constant_values=N << 14).reshape(NS, NCHUNK, CHUNK)
    zslab = jnp.zeros((RPT, HALF), jnp.float32)

    partials = _sc_deg(row, col)
    norms = _tc_prep(partials)
    nt = jnp.transpose(norms[:, :N])

    hs = _tc_pre(x, nt)
    for _ in range(K_APPNP - 1):
        acc = _sc_prop(hs, pki, zslab)
        hs = _tc_mid(acc, x, nt)
    acc = _sc_prop(hs, pki, zslab)
    h = _tc_last(acc, x, nt)

    acc = _sc_prop(h, pki, zslab)
    h = _tc_cgc0(acc, h, nt, W_out0, b_out0, W_root0)
    acc = _sc_prop(h, pki, zslab)
    return _tc_cgc1(acc, h, nt, W_out1, b_out1, W_root1)


# speedup vs baseline: 2.4056x; 1.0996x over previous
import jax
import jax.numpy as jnp
from jax import lax
from jax.experimental import pallas as pl
from jax.experimental.pallas import tpu as pltpu
from jax.experimental.pallas import tpu_sc as plsc

N = 10000
D = 128
E = 320000
K_APPNP = 5
ALPHA = 0.2

NC = 2
NS = 16
L = 16
HALF = D // NC

NP = 10112
RPT = NP // NS

EPT = E // NS
CHUNK = 112
NBUF = 4
NCHUNK = 180
EPTP = NCHUNK * CHUNK

EPW = E // (NC * NS)

_MESH = plsc.VectorSubcoreMesh(
    core_axis_name="c", subcore_axis_name="s", num_cores=NC, num_subcores=NS
)
_SC_PARAMS = pltpu.CompilerParams(
    use_tc_tiling_on_sc=False, needs_layout_passes=False
)


def _sc_deg_body(row_hbm, col_hbm, out_hbm, row_v, col_v, hist):
    c = lax.axis_index("c")
    s = lax.axis_index("s")
    w = s * NC + c
    base = w * EPW
    pltpu.sync_copy(row_hbm.at[pl.ds(base, EPW)], row_v)
    pltpu.sync_copy(col_hbm.at[pl.ds(base, EPW)], col_v)

    zeros = jnp.zeros((L,), jnp.float32)

    def zb(i, carry):
        for j in range(3):
            hist[j, pl.ds(i * L, L)] = zeros
        return carry

    lax.fori_loop(0, NP // L, zb, 0)

    ones = jnp.ones((L,), jnp.float32)
    r0 = jnp.zeros((L,), jnp.int32)
    r1 = jnp.full((L,), 1, jnp.int32)
    r2 = jnp.full((L,), 2, jnp.int32)

    def eb(i, carry):
        r = row_v[pl.ds(i * L, L)]
        cl = col_v[pl.ds(i * L, L)]
        plsc.addupdate_scatter(hist, [r0, r], ones)
        plsc.addupdate_scatter(hist, [r1, cl], ones)
        plsc.addupdate_scatter(hist, [r2, cl], ones, mask=r == cl)
        return carry

    lax.fori_loop(0, EPW // L, eb, 0)

    pltpu.sync_copy(hist, out_hbm.at[pl.ds(w * 8, 8)])


_sc_deg = pl.kernel(
    _sc_deg_body,
    out_type=jax.ShapeDtypeStruct((NC * NS * 8, NP), jnp.float32),
    mesh=_MESH,
    scratch_types=[
        pltpu.VMEM((EPW,), jnp.int32),
        pltpu.VMEM((EPW,), jnp.int32),
        pltpu.VMEM((8, NP), jnp.float32),
    ],
    compiler_params=_SC_PARAMS,
)


def _sc_prop_body(h_hbm, pki_hbm, zslab_hbm, out_hbm,
                  h_sp, acc_sh, pk_v, rb0, cb0, rb1, cb1, rb2, cb2, rb3, cb3,
                  gb0, gb1, gb2, gb3, sg0, sg1, sg2, sg3, ss0, ss1, ss2, ss3):
    c = lax.axis_index("c")
    s = lax.axis_index("s")
    rbufs = (rb0, rb1, rb2, rb3)
    cbufs = (cb0, cb1, cb2, cb3)
    gbufs = (gb0, gb1, gb2, gb3)
    semg = (sg0, sg1, sg2, sg3)
    sems = (ss0, ss1, ss2, ss3)

    pltpu.sync_copy(pki_hbm.at[s], pk_v)
    pltpu.sync_copy(h_hbm.at[pl.ds(s * RPT, RPT), pl.ds(c * HALF, HALF)],
                    h_sp.at[pl.ds(s * RPT, RPT)])
    pltpu.sync_copy(zslab_hbm, acc_sh.at[pl.ds(s * RPT, RPT)])
    plsc.subcore_barrier()

    mask14 = jnp.full((L,), 0x3FFF, jnp.int32)
    sh14 = jnp.full((L,), 14, jnp.int32)

    def unpack(g, b):
        for i in range(CHUNK // L):
            v = pk_v[g, pl.ds(i * L, L)]
            rbufs[b][pl.ds(i * L, L)] = v & mask14
            cbufs[b][pl.ds(i * L, L)] = lax.shift_right_logical(v, sh14)

    def body(q, carry):
        g = NBUF * q
        cps = []
        for b in range(NBUF):
            unpack(g + b, b)
            cps.append(pltpu.async_copy(h_sp.at[rbufs[b]], gbufs[b], semg[b]))
        scs = []
        for b in range(NBUF):
            cps[b].wait()
            scs.append(
                pltpu.async_copy(gbufs[b], acc_sh.at[cbufs[b]],
                                 sems[b], add=True)
            )
        for b in range(NBUF):
            scs[b].wait()
        return carry

    lax.fori_loop(0, NCHUNK // NBUF, body, 0)
    plsc.subcore_barrier()

    pltpu.sync_copy(
        acc_sh.at[pl.ds(s * RPT, RPT)],
        out_hbm.at[pl.ds(s * RPT, RPT), pl.ds(c * HALF, HALF)],
    )


_sc_prop = pl.kernel(
    _sc_prop_body,
    out_type=jax.ShapeDtypeStruct((NP, D), jnp.float32),
    mesh=_MESH,
    scratch_types=[
        pltpu.VMEM_SHARED((NP, HALF), jnp.float32),
        pltpu.VMEM_SHARED((NP, HALF), jnp.float32),
        pltpu.VMEM((NCHUNK, CHUNK), jnp.int32),
        pltpu.VMEM((CHUNK,), jnp.int32),
        pltpu.VMEM((CHUNK,), jnp.int32),
        pltpu.VMEM((CHUNK,), jnp.int32),
        pltpu.VMEM((CHUNK,), jnp.int32),
        pltpu.VMEM((CHUNK,), jnp.int32),
        pltpu.VMEM((CHUNK,), jnp.int32),
        pltpu.VMEM((CHUNK,), jnp.int32),
        pltpu.VMEM((CHUNK,), jnp.int32),
        pltpu.VMEM((CHUNK, HALF), jnp.float32),
        pltpu.VMEM((CHUNK, HALF), jnp.float32),
        pltpu.VMEM((CHUNK, HALF), jnp.float32),
        pltpu.VMEM((CHUNK, HALF), jnp.float32),
        pltpu.SemaphoreType.DMA,
        pltpu.SemaphoreType.DMA,
        pltpu.SemaphoreType.DMA,
        pltpu.SemaphoreType.DMA,
        pltpu.SemaphoreType.DMA,
        pltpu.SemaphoreType.DMA,
        pltpu.SemaphoreType.DMA,
        pltpu.SemaphoreType.DMA,
    ],
    compiler_params=_SC_PARAMS,
)


def _tc_prep_body(part_ref, norms_ref):
    ps = jnp.sum(part_ref[...].reshape(NC * NS, 8, NP)[:, :3], axis=0)
    deg_out = ps[0]
    deg_in = ps[1]
    selfc = ps[2]
    src = lax.rsqrt(jnp.maximum(deg_out, 1.0))
    dst_eff = (1.0 - ALPHA) * lax.rsqrt(jnp.maximum(deg_in, 1.0))
    cgc_inv = 1.0 / jnp.maximum(deg_in - selfc + 1.0, 1.0)
    norms_ref[...] = jnp.stack([src, dst_eff, cgc_inv, selfc])


_tc_prep = pl.pallas_call(
    _tc_prep_body,
    out_shape=jax.ShapeDtypeStruct((4, NP), jnp.float32),
)


def _tc_pre_body(x_ref, nt_ref, hs_ref):
    hs_ref[0:N, :] = x_ref[...] * nt_ref[:, 0:1]


_tc_pre = pl.pallas_call(
    _tc_pre_body,
    out_shape=jax.ShapeDtypeStruct((NP, D), jnp.float32),
)


def _tc_mid_body(acc_ref, x0_ref, nt_ref, hs_ref):
    h = acc_ref[0:N, :] * nt_ref[:, 1:2] + ALPHA * x0_ref[...]
    hs_ref[0:N, :] = h * nt_ref[:, 0:1]


_tc_mid = pl.pallas_call(
    _tc_mid_body,
    out_shape=jax.ShapeDtypeStruct((NP, D), jnp.float32),
)


def _tc_last_body(acc_ref, x0_ref, nt_ref, h_ref):
    h_ref[0:N, :] = acc_ref[0:N, :] * nt_ref[:, 1:2] + ALPHA * x0_ref[...]


_tc_last = pl.pallas_call(
    _tc_last_body,
    out_shape=jax.ShapeDtypeStruct((NP, D), jnp.float32),
)


def _mish(z):
    return z * jnp.tanh(jax.nn.softplus(z))


def _cgc(acc_ref, h_ref, nt_ref, wo_ref, bo_ref, wr_ref):
    h = h_ref[0:N, :]
    agg = nt_ref[:, 2:3] * (acc_ref[0:N, :] + (1.0 - nt_ref[:, 3:4]) * h)
    return (jnp.dot(agg, wo_ref[...], preferred_element_type=jnp.float32)
            + bo_ref[...][None, :]
            + jnp.dot(h, wr_ref[...], preferred_element_type=jnp.float32))


def _tc_cgc0_body(acc_ref, h_ref, nt_ref, wo_ref, bo_ref, wr_ref, out_ref):
    out_ref[0:N, :] = _cgc(acc_ref, h_ref, nt_ref, wo_ref, bo_ref, wr_ref)


_tc_cgc0 = pl.pallas_call(
    _tc_cgc0_body,
    out_shape=jax.ShapeDtypeStruct((NP, D), jnp.float32),
)


def _tc_cgc1_body(acc_ref, h_ref, nt_ref, wo_ref, bo_ref, wr_ref, out_ref):
    out_ref[...] = _mish(_cgc(acc_ref, h_ref, nt_ref, wo_ref, bo_ref, wr_ref))


_tc_cgc1 = pl.pallas_call(
    _tc_cgc1_body,
    out_shape=jax.ShapeDtypeStruct((N, D), jnp.float32),
)


def kernel(x, edge_index, W_out0, b_out0, W_root0, W_out1, b_out1, W_root1):
    row = edge_index[0].astype(jnp.int32)
    col = edge_index[1].astype(jnp.int32)

    packed = row | (col << 14)
    pki = jnp.pad(packed.reshape(NS, EPT), ((0, 0), (0, EPTP - EPT)),
                  constant_values=N << 14).reshape(NS, NCHUNK, CHUNK)
    zslab = jnp.zeros((RPT, HALF), jnp.float32)

    partials = _sc_deg(row, col)
    norms = _tc_prep(partials)
    nt = jnp.transpose(norms[:, :N])

    hs = _tc_pre(x, nt)
    for _ in range(K_APPNP - 1):
        acc = _sc_prop(hs, pki, zslab)
        hs = _tc_mid(acc, x, nt)
    acc = _sc_prop(hs, pki, zslab)
    h = _tc_last(acc, x, nt)

    acc = _sc_prop(h, pki, zslab)
    h = _tc_cgc0(acc, h, nt, W_out0, b_out0, W_root0)
    acc = _sc_prop(h, pki, zslab)
    return _tc_cgc1(acc, h, nt, W_out1, b_out1, W_root1)
